# Initial kernel scaffold; baseline (speedup 1.0000x reference)
import jax
import jax.numpy as jnp
from jax.experimental import pallas as pl


def kernel(x, edge_index, W1, b1, ln1_g, ln1_b, W2, b2, ln2_g, ln2_b, Wp, bp):
    raise NotImplementedError("write your pallas kernel here")


# speedup vs baseline: 22.1643x; 22.1643x over previous
import functools

import jax
import jax.numpy as jnp
from jax import lax
from jax.experimental import pallas as pl
from jax.experimental.pallas import tpu as pltpu
from jax.experimental.pallas import tpu_sc as plsc

N = 10000
E = 320000
D_IN = 128
DH = 16
D_OUT = 128

NC, NS = 2, 16
NW = NC * NS
B = 100
IDX_ROWS = E // B // NW
NPAD = 10240
N_PER_SUB = NPAD // NS

_MESH = plsc.VectorSubcoreMesh(
    core_axis_name="c", subcore_axis_name="s", num_cores=NC, num_subcores=NS)


@functools.partial(
    pl.kernel,
    out_type=jax.ShapeDtypeStruct((NC, NPAD, DH), jnp.float32),
    mesh=_MESH,
    scratch_types=[
        pltpu.VMEM((IDX_ROWS, B), jnp.int32),
        pltpu.VMEM((IDX_ROWS, B), jnp.int32),
        pltpu.VMEM((B, DH), jnp.float32),
        pltpu.VMEM((N_PER_SUB, DH), jnp.float32),
        pltpu.VMEM_SHARED((NPAD, DH), jnp.float32),
        pltpu.SemaphoreType.DMA,
    ],
    compiler_params=pltpu.CompilerParams(use_tc_tiling_on_sc=False),
)
def _edge_pass(table, srcb, dstb, out, sidx, didx, rows, acc, shared, sem):
    c = lax.axis_index("c")
    s = lax.axis_index("s")
    wid = c * NS + s

    def _zrow(i, carry):
        acc[i, :] = jnp.zeros((DH,), jnp.float32)
        return carry
    lax.fori_loop(0, N_PER_SUB, _zrow, 0)
    pltpu.sync_copy(acc, shared.at[pl.ds(s * N_PER_SUB, N_PER_SUB)])

    pltpu.sync_copy(srcb.at[wid], sidx)
    pltpu.sync_copy(dstb.at[wid], didx)
    plsc.subcore_barrier()

    def _batch(g, carry):
        pltpu.async_copy(table.at[sidx.at[g]], rows, sem).wait()
        pltpu.sync_copy(rows, shared.at[didx.at[g]], add=True)
        return carry
    lax.fori_loop(0, IDX_ROWS, _batch, 0)

    plsc.subcore_barrier()
    pltpu.sync_copy(shared.at[pl.ds(s * N_PER_SUB, N_PER_SUB)], acc)
    pltpu.sync_copy(acc, out.at[c, pl.ds(s * N_PER_SUB, N_PER_SUB)])


def _mm_body(x_ref, w_ref, o_ref):
    o_ref[...] = jnp.dot(x_ref[...], w_ref[...],
                         preferred_element_type=jnp.float32)


def _matmul_xw1(x, w):
    R = 1000
    return pl.pallas_call(
        _mm_body,
        grid=(N // R,),
        in_specs=[pl.BlockSpec((R, D_IN), lambda i: (i, 0)),
                  pl.BlockSpec((D_IN, DH), lambda i: (0, 0))],
        out_specs=pl.BlockSpec((R, DH), lambda i: (i, 0)),
        out_shape=jax.ShapeDtypeStruct((N, DH), jnp.float32),
    )(x, w)


def _prep_body(degp_ref, m1_ref, m1s_o, dinv_o, dinv2_o):
    deg = degp_ref[0] + degp_ref[1] + 1.0
    dinv = lax.rsqrt(deg)
    dinv_o[...] = dinv
    dinv2_o[...] = 1.0 / deg
    m1s_o[...] = m1_ref[...] * dinv


def _prep(deg_p, m1):
    return pl.pallas_call(
        _prep_body,
        out_shape=[jax.ShapeDtypeStruct((N, DH), jnp.float32)] * 3,
    )(deg_p, m1)


def _ln_relu(sval, g, b):
    mu = jnp.mean(sval, axis=-1, keepdims=True)
    var = jnp.mean((sval - mu) * (sval - mu), axis=-1, keepdims=True)
    return jnp.maximum((sval - mu) * lax.rsqrt(var + 1e-5) * g + b, 0.0)


def _post1_body(p_ref, m_ref, dinv_ref, dinv2_ref, b_ref, g_ref, be_ref,
                w2_ref, h_o, m2_o, m2s_o):
    sval = ((p_ref[0] + p_ref[1]) * dinv_ref[...]
            + m_ref[...] * dinv2_ref[...] + b_ref[...])
    h = _ln_relu(sval, g_ref[...], be_ref[...])
    h_o[...] = h
    m2 = jnp.dot(h, w2_ref[...], preferred_element_type=jnp.float32)
    m2_o[...] = m2
    m2s_o[...] = m2 * dinv_ref[...]


def _post1(p1, m1, dinv, dinv2, b1, g1, be1, w2):
    return pl.pallas_call(
        _post1_body,
        out_shape=[jax.ShapeDtypeStruct((N, DH), jnp.float32)] * 3,
    )(p1, m1, dinv, dinv2, b1, g1, be1, w2)


def _final_body(p_ref, m2_ref, dinv_ref, dinv2_ref, b_ref, g_ref, be_ref,
                x_ref, h1_ref, wx_ref, wh1_ref, wh2_ref, bp_ref, o_ref):
    sval = ((p_ref[0] + p_ref[1]) * dinv_ref[...]
            + m2_ref[...] * dinv2_ref[...] + b_ref[...])
    h2 = _ln_relu(sval, g_ref[...], be_ref[...])
    acc = jnp.dot(x_ref[...], wx_ref[...], preferred_element_type=jnp.float32)
    acc += jnp.dot(h1_ref[...], wh1_ref[...], preferred_element_type=jnp.float32)
    acc += jnp.dot(h2, wh2_ref[...], preferred_element_type=jnp.float32)
    o_ref[...] = acc + bp_ref[...]


def _final(p2, m2, dinv, dinv2, b2, g2, be2, x, h1, wx, wh1, wh2, bp):
    R = 1000
    small = pl.BlockSpec((R, DH), lambda i: (i, 0))
    row16 = pl.BlockSpec((1, DH), lambda i: (0, 0))
    return pl.pallas_call(
        _final_body,
        grid=(N // R,),
        in_specs=[
            pl.BlockSpec((NC, R, DH), lambda i: (0, i, 0)),
            small, small, small,
            row16, row16, row16,
            pl.BlockSpec((R, D_IN), lambda i: (i, 0)),
            small,
            pl.BlockSpec((D_IN, D_OUT), lambda i: (0, 0)),
            pl.BlockSpec((DH, D_OUT), lambda i: (0, 0)),
            pl.BlockSpec((DH, D_OUT), lambda i: (0, 0)),
            pl.BlockSpec((1, D_OUT), lambda i: (0, 0)),
        ],
        out_specs=pl.BlockSpec((R, D_OUT), lambda i: (i, 0)),
        out_shape=jax.ShapeDtypeStruct((N, D_OUT), jnp.float32),
    )(p2, m2, dinv, dinv2, b2, g2, be2, x, h1, wx, wh1, wh2, bp)


def kernel(x, edge_index, W1, b1, ln1_g, ln1_b, W2, b2, ln2_g, ln2_b, Wp, bp):
    src = edge_index[0].reshape(NW, IDX_ROWS, B)
    dst = edge_index[1].reshape(NW, IDX_ROWS, B)

    b1r, g1r, be1r = (a.reshape(1, DH) for a in (b1, ln1_g, ln1_b))
    b2r, g2r, be2r = (a.reshape(1, DH) for a in (b2, ln2_g, ln2_b))
    bpr = bp.reshape(1, D_OUT)
    wx, wh1, wh2 = Wp[:D_IN], Wp[D_IN:D_IN + DH], Wp[D_IN + DH:]

    ones_t = jnp.ones((N, DH), jnp.float32)
    deg_p = _edge_pass(ones_t, dst, dst)[:, :N]
    m1 = _matmul_xw1(x, W1)
    m1s, dinv, dinv2 = _prep(deg_p, m1)
    p1 = _edge_pass(m1s, src, dst)[:, :N]
    h1, m2, m2s = _post1(p1, m1, dinv, dinv2, b1r, g1r, be1r, W2)
    p2 = _edge_pass(m2s, src, dst)[:, :N]
    return _final(p2, m2, dinv, dinv2, b2r, g2r, be2r, x, h1,
                  wx, wh1, wh2, bpr)
